# baseline (speedup 1.0000x reference)
import jax
import jax.numpy as jnp
import numpy as np
from jax import lax
from jax.experimental import pallas as pl
from jax.experimental.pallas import tpu as pltpu
from jax.experimental.pallas import tpu_sc as plsc

N = 800000
S = 50000

_T = np.float32(0.1)
_ALPHA2 = np.clip((np.float32(1.0) - _T * _T) ** np.float32(2.0), np.float32(1e-5), np.float32(1.0))
_ALPHA = np.sqrt(_ALPHA2).astype(np.float32)
_SIGMA = np.sqrt(np.float32(1.0) - _ALPHA2).astype(np.float32)

_NC, _NS = 2, 16
_NW = _NC * _NS
_RPW = N // _NW
_CHUNK = 1000
_NCH = _RPW // _CHUNK
_RIT = -(-_CHUNK // 16)

_SPAD = 50048
_TS = _SPAD // _NS
_TW = _SPAD // _NW

_BLK = 8192
_NBLK = -(-N // _BLK)
_NPAD = _NBLK * _BLK
_RPWC = _NPAD // _NW
_CHUNKC = 1568
_NCHC = _RPWC // _CHUNKC

_MESH = plsc.VectorSubcoreMesh(core_axis_name="c", subcore_axis_name="s")
_SC_PARAMS = pltpu.CompilerParams(needs_layout_passes=False,
                                  use_tc_tiling_on_sc=False)


def _stats_body(mol_hbm, eps_hbm, idx_hbm, out_hbm,
                xbuf, ebuf, ibuf, i8buf, cxbuf, onebuf, tbuf, table):
    cidx = lax.axis_index("c")
    sidx = lax.axis_index("s")
    w = sidx * _NC + cidx
    t0 = sidx * _TS * 8
    iota = lax.iota(jnp.int32, 16)
    ones16 = jnp.full((16,), 1.0, jnp.float32)
    zer16 = jnp.full((16,), 0.0, jnp.float32)

    def fill_ones(j, carry):
        onebuf[pl.ds(j * 16, 16)] = ones16
        return carry

    lax.fori_loop(0, _CHUNK // 16 + 1, fill_ones, 0)

    def fill_zeros(j, carry):
        tbuf[pl.ds(j * 16, 16)] = zer16
        return carry

    lax.fori_loop(0, _TS * 8 // 16, fill_zeros, 0)
    pltpu.sync_copy(tbuf, table.at[pl.ds(t0, _TS * 8)])
    plsc.subcore_barrier()

    def chunk_body(k, carry):
        r0 = w * _RPW + k * _CHUNK
        pltpu.sync_copy(mol_hbm.at[pl.ds(r0 * 19, _CHUNK * 19)], xbuf)
        pltpu.sync_copy(eps_hbm.at[pl.ds(r0 * 3, _CHUNK * 3)], ebuf)
        pltpu.sync_copy(idx_hbm.at[pl.ds(r0, _CHUNK)], ibuf)

        def row_body(j, carry2):
            st = jnp.minimum(j * 16, _CHUNK - 16)
            rows = st + iota
            iv8 = ibuf[pl.ds(st, 16)] * 8
            i8buf[pl.ds(st, 16)] = iv8
            for col in range(3):
                v = plsc.load_gather(xbuf, [rows * 19 + col])
                cxbuf[pl.ds(col * _CHUNK + st, 16)] = v
            for col in range(3):
                v = plsc.load_gather(ebuf, [rows * 3 + col])
                cxbuf[pl.ds((col + 3) * _CHUNK + st, 16)] = v
            return carry2

        lax.fori_loop(0, _RIT, row_body, 0)

        for col in range(7):
            if col > 0:
                def rebuild(j, carry2):
                    st = jnp.minimum(j * 16, _CHUNK - 16)
                    i8buf[pl.ds(st, 16)] = ibuf[pl.ds(st, 16)] * 8 + col
                    return carry2

                lax.fori_loop(0, _RIT, rebuild, 0)
            if col < 6:
                pltpu.sync_copy(cxbuf.at[pl.ds(col * _CHUNK, _CHUNK)],
                                table.at[i8buf.at[...]], add=True)
            else:
                pltpu.sync_copy(onebuf, table.at[i8buf.at[...]], add=True)
        return carry

    lax.fori_loop(0, _NCH, chunk_body, 0)
    plsc.subcore_barrier()
    pltpu.sync_copy(table.at[pl.ds(t0, _TS * 8)], tbuf)
    pltpu.sync_copy(tbuf, out_hbm.at[pl.ds(cidx * _SPAD * 8 + t0, _TS * 8)])


_stats = pl.kernel(
    _stats_body,
    out_type=jax.ShapeDtypeStruct((2 * _SPAD * 8,), jnp.float32),
    mesh=_MESH,
    compiler_params=_SC_PARAMS,
    scratch_types=[
        pltpu.VMEM((_CHUNK * 19,), jnp.float32),
        pltpu.VMEM((_CHUNK * 3,), jnp.float32),
        pltpu.VMEM((_CHUNK,), jnp.int32),
        pltpu.VMEM((_CHUNK,), jnp.int32),
        pltpu.VMEM((6 * _CHUNK,), jnp.float32),
        pltpu.VMEM((_CHUNK,), jnp.float32),
        pltpu.VMEM((_TS * 8,), jnp.float32),
        pltpu.VMEM_SHARED((_SPAD * 8,), jnp.float32),
    ],
)


def _combine_body(parts_hbm, out_hbm, pa, pb, cbuf):
    cidx = lax.axis_index("c")
    sidx = lax.axis_index("s")
    w = sidx * _NC + cidx
    o0 = w * _TW * 8
    pltpu.sync_copy(parts_hbm.at[pl.ds(o0, _TW * 8)], pa)
    pltpu.sync_copy(parts_hbm.at[pl.ds(_SPAD * 8 + o0, _TW * 8)], pb)
    iota = lax.iota(jnp.int32, 16)

    def comb_body(j, carry):
        st = jnp.minimum(j * 16, _TW - 16)
        rows = (st + iota) * 8
        cnt = (plsc.load_gather(pa, [rows + 6])
               + plsc.load_gather(pb, [rows + 6]))
        inv = jnp.float32(1.0) / jnp.maximum(cnt, 1.0)
        for col in range(3):
            v = (plsc.load_gather(pa, [rows + col])
                 + plsc.load_gather(pb, [rows + col])) * inv
            plsc.store_scatter(cbuf, [rows + col], v)
        for col in range(3, 6):
            v = (plsc.load_gather(pa, [rows + col])
                 + plsc.load_gather(pb, [rows + col])) * (inv * _SIGMA)
            plsc.store_scatter(cbuf, [rows + col], v)
        plsc.store_scatter(cbuf, [rows + 6], inv)
        plsc.store_scatter(cbuf, [rows + 7], jnp.full((16,), 0.0, jnp.float32))
        return carry

    lax.fori_loop(0, -(-_TW // 16), comb_body, 0)
    pltpu.sync_copy(cbuf, out_hbm.at[pl.ds(o0, _TW * 8)])


_combine = pl.kernel(
    _combine_body,
    out_type=jax.ShapeDtypeStruct((_SPAD * 8,), jnp.float32),
    mesh=_MESH,
    compiler_params=_SC_PARAMS,
    scratch_types=[
        pltpu.VMEM((_TW * 8,), jnp.float32),
        pltpu.VMEM((_TW * 8,), jnp.float32),
        pltpu.VMEM((_TW * 8,), jnp.float32),
    ],
)


def _gather_body(tbl_hbm, idx_hbm, out_hbm, ibuf, gbuf, sbuf, table, sem):
    cidx = lax.axis_index("c")
    sidx = lax.axis_index("s")
    w = sidx * _NC + cidx
    t0 = sidx * _TS
    pltpu.sync_copy(tbl_hbm.at[pl.ds(t0, _TS)], sbuf)
    pltpu.sync_copy(sbuf, table.at[pl.ds(t0, _TS)])
    plsc.subcore_barrier()

    def chunk_body(k, carry):
        r0 = w * _RPW + k * _CHUNK
        pltpu.sync_copy(idx_hbm.at[pl.ds(r0, _CHUNK)], ibuf)
        pltpu.async_copy(table.at[ibuf.at[...]], gbuf, sem).wait()
        pltpu.sync_copy(gbuf, out_hbm.at[pl.ds(r0, _CHUNK)])
        return carry

    lax.fori_loop(0, _NCH, chunk_body, 0)


_gather = pl.kernel(
    _gather_body,
    out_type=jax.ShapeDtypeStruct((N, 8), jnp.float32),
    mesh=_MESH,
    compiler_params=_SC_PARAMS,
    scratch_types=[
        pltpu.VMEM((_CHUNK,), jnp.int32),
        pltpu.VMEM((_CHUNK, 8), jnp.float32),
        pltpu.VMEM((_TS, 8), jnp.float32),
        pltpu.VMEM_SHARED((_SPAD, 8), jnp.float32),
        pltpu.SemaphoreType.DMA,
    ],
)


def _dense_body(mol_ref, ex_ref, eh_ref, g_ref, w1_ref, w2_ref, r_ref, loss_ref):
    x3 = mol_ref[:, 0:3]
    xh = mol_ref[:, 3:19]
    com = g_ref[:, 0:3]
    m2 = g_ref[:, 3:6]
    invc = g_ref[:, 6:7]
    ex = ex_ref[...]
    tx = x3 - com
    zx = _ALPHA * tx + _SIGMA * ex - m2
    zh = (_ALPHA * np.float32(0.25)) * xh + _SIGMA * eh_ref[...]
    z = jnp.concatenate([zx, zh], axis=1)
    h = jnp.maximum(jnp.dot(z, w1_ref[...], preferred_element_type=jnp.float32), 0.0)
    e = jnp.dot(h, w2_ref[...], preferred_element_type=jnp.float32)
    e3 = e[:, 0:3]
    zhat = (np.float32(1.0) / _ALPHA) * zx - (_SIGMA / _ALPHA) * e3
    dr = tx - zhat
    r = jnp.sum(dr * dr, axis=1) * (invc[:, 0] * np.float32(1.0 / 3.0))
    dl = ex - e3
    l = jnp.sum(dl * dl, axis=1) * invc[:, 0]
    i = pl.program_id(0)
    valid = i * _BLK + lax.iota(jnp.int32, _BLK) < N
    r_ref[...] = jnp.where(valid, r, 0.0)
    l = jnp.where(valid, l, 0.0)

    @pl.when(i == 0)
    def _init():
        loss_ref[...] = jnp.zeros((1, 1), jnp.float32)

    loss_ref[...] += jnp.sum(l)[None, None]


def _dense_pass(mol_x, eps_x, eps_h, g, W1, W2):
    return pl.pallas_call(
        _dense_body,
        grid=(_NBLK,),
        in_specs=[
            pl.BlockSpec((_BLK, 19), lambda i: (i, 0)),
            pl.BlockSpec((_BLK, 3), lambda i: (i, 0)),
            pl.BlockSpec((_BLK, 16), lambda i: (i, 0)),
            pl.BlockSpec((_BLK, 8), lambda i: (i, 0)),
            pl.BlockSpec((19, 64), lambda i: (0, 0)),
            pl.BlockSpec((64, 19), lambda i: (0, 0)),
        ],
        out_specs=[
            pl.BlockSpec((_BLK,), lambda i: (i,)),
            pl.BlockSpec((1, 1), lambda i: (0, 0)),
        ],
        out_shape=[
            jax.ShapeDtypeStruct((_NPAD,), jnp.float32),
            jax.ShapeDtypeStruct((1, 1), jnp.float32),
        ],
    )(mol_x, eps_x, eps_h, g, W1, W2)


def _rscatter_body(r_hbm, idxp_hbm, out_hbm, rbuf, ibuf, zbuf, table):
    cidx = lax.axis_index("c")
    sidx = lax.axis_index("s")
    w = sidx * _NC + cidx
    t0 = sidx * _TS
    zer16 = jnp.full((16,), 0.0, jnp.float32)

    def fill_zeros(j, carry):
        zbuf[pl.ds(j * 16, 16)] = zer16
        return carry

    lax.fori_loop(0, _TS // 16, fill_zeros, 0)
    pltpu.sync_copy(zbuf, table.at[pl.ds(t0, _TS)])
    plsc.subcore_barrier()

    def chunk_body(k, carry):
        r0 = w * _RPWC + k * _CHUNKC
        pltpu.sync_copy(r_hbm.at[pl.ds(r0, _CHUNKC)], rbuf)
        pltpu.sync_copy(idxp_hbm.at[pl.ds(r0, _CHUNKC)], ibuf)
        pltpu.sync_copy(rbuf, table.at[ibuf.at[...]], add=True)
        return carry

    lax.fori_loop(0, _NCHC, chunk_body, 0)
    plsc.subcore_barrier()
    pltpu.sync_copy(table.at[pl.ds(t0, _TS)], zbuf)
    pltpu.sync_copy(zbuf, out_hbm.at[pl.ds(cidx * _SPAD + t0, _TS)])


_rscatter = pl.kernel(
    _rscatter_body,
    out_type=jax.ShapeDtypeStruct((2 * _SPAD,), jnp.float32),
    mesh=_MESH,
    compiler_params=_SC_PARAMS,
    scratch_types=[
        pltpu.VMEM((_CHUNKC,), jnp.float32),
        pltpu.VMEM((_CHUNKC,), jnp.int32),
        pltpu.VMEM((_TS,), jnp.float32),
        pltpu.VMEM_SHARED((_SPAD,), jnp.float32),
    ],
)


def _final_body(rp_ref, loss_ref, lo_ref, ro_ref):
    rsum = rp_ref[pl.ds(0, _SPAD)] + rp_ref[pl.ds(_SPAD, _SPAD)]
    ro_ref[...] = (jnp.sum(jnp.sqrt(rsum)) * np.float32(1.0 / S))[None, None]
    lo_ref[...] = loss_ref[...] * np.float32(1.0 / (6.0 * S))


def _final(rparts, loss_acc):
    return pl.pallas_call(
        _final_body,
        out_shape=[
            jax.ShapeDtypeStruct((1, 1), jnp.float32),
            jax.ShapeDtypeStruct((1, 1), jnp.float32),
        ],
    )(rparts, loss_acc)


_ABLATE = 1


def kernel(mol_x, mol_idx, pro_x, pro_idx, eps_x_mol, eps_h_mol, eps_h_pro, W1m, W2m, W1p, W2p):
    idx = mol_idx.astype(jnp.int32)
    idx_pad = jnp.concatenate([idx, jnp.zeros((_NPAD - N,), jnp.int32)])
    mol_flat = mol_x.reshape(N * 19)
    eps_flat = eps_x_mol.reshape(N * 3)
    parts = _stats(mol_flat, eps_flat, idx_pad)
    if _ABLATE == 1:
        return (parts[0], parts[1])
    tbl = _combine(parts)
    g = _gather(tbl.reshape(_SPAD, 8), idx_pad)
    r, loss_acc = _dense_pass(mol_x, eps_x_mol, eps_h_mol, g, W1m, W2m)
    rparts = _rscatter(r, idx_pad)
    lo, ro = _final(rparts, loss_acc)
    return (lo[0, 0], ro[0, 0])


# speedup vs baseline: 1.6526x; 1.6526x over previous
import jax
import jax.numpy as jnp
import numpy as np
from jax import lax
from jax.experimental import pallas as pl
from jax.experimental.pallas import tpu as pltpu
from jax.experimental.pallas import tpu_sc as plsc

N = 800000
S = 50000

_T = np.float32(0.1)
_ALPHA2 = np.clip((np.float32(1.0) - _T * _T) ** np.float32(2.0), np.float32(1e-5), np.float32(1.0))
_ALPHA = np.sqrt(_ALPHA2).astype(np.float32)
_SIGMA = np.sqrt(np.float32(1.0) - _ALPHA2).astype(np.float32)

_NC, _NS = 2, 16
_NW = _NC * _NS
_RPW = N // _NW
_CHUNK = 1000
_NCH = _RPW // _CHUNK
_CHUNKA = 5000
_NCHA = _RPW // _CHUNKA

_SPAD = 50176
_TS = _SPAD // _NS
_TW = _SPAD // _NW

_BLK = 8192
_NBLK = -(-N // _BLK)
_NPAD = _NBLK * _BLK
_RPWC = _NPAD // _NW
_CHUNKC = 1568
_NCHC = _RPWC // _CHUNKC

_MESH = plsc.VectorSubcoreMesh(core_axis_name="c", subcore_axis_name="s")
_SC_PARAMS = pltpu.CompilerParams(needs_layout_passes=False,
                                  use_tc_tiling_on_sc=False)


def _stats_body(x0_hbm, x1_hbm, x2_hbm, e0_hbm, e1_hbm, e2_hbm, idx_hbm,
                out_hbm, cbufs, ibuf, onebuf, tbuf, table):
    cidx = lax.axis_index("c")
    sidx = lax.axis_index("s")
    w = sidx * _NC + cidx
    t0 = sidx * _TS * 7
    iota = lax.iota(jnp.int32, 16)
    ones16 = jnp.full((16,), 1.0, jnp.float32)
    zer16 = jnp.full((16,), 0.0, jnp.float32)

    def fill_ones(j, carry):
        onebuf[pl.ds(j * 16, 16)] = ones16
        return carry

    lax.fori_loop(0, _CHUNKA // 16 + 1, fill_ones, 0)

    def fill_zeros(j, carry):
        tbuf[pl.ds(j * 16, 16)] = zer16
        return carry

    lax.fori_loop(0, _TS * 7 // 16, fill_zeros, 0)
    pltpu.sync_copy(tbuf, table.at[pl.ds(t0, _TS * 7)])
    plsc.subcore_barrier()
    cols = (x0_hbm, x1_hbm, x2_hbm, e0_hbm, e1_hbm, e2_hbm)

    def chunk_body(k, carry):
        r0 = w * _RPW + k * _CHUNKA
        for c in range(6):
            pltpu.sync_copy(cols[c].at[pl.ds(r0, _CHUNKA)],
                            cbufs.at[pl.ds(c * _CHUNKA, _CHUNKA)])
        pltpu.sync_copy(idx_hbm.at[pl.ds(r0, _CHUNKA)], ibuf)
        for c in range(6):
            pltpu.sync_copy(cbufs.at[pl.ds(c * _CHUNKA, _CHUNKA)],
                            table.at[pl.ds(c * _SPAD, _SPAD)].at[ibuf.at[...]],
                            add=True)
        pltpu.sync_copy(onebuf,
                        table.at[pl.ds(6 * _SPAD, _SPAD)].at[ibuf.at[...]],
                        add=True)
        return carry

    lax.fori_loop(0, _NCHA, chunk_body, 0)
    plsc.subcore_barrier()
    pltpu.sync_copy(table.at[pl.ds(t0, _TS * 7)], tbuf)
    pltpu.sync_copy(tbuf, out_hbm.at[pl.ds(cidx * _SPAD * 7 + t0, _TS * 7)])


_stats = pl.kernel(
    _stats_body,
    out_type=jax.ShapeDtypeStruct((2 * _SPAD * 7,), jnp.float32),
    mesh=_MESH,
    compiler_params=_SC_PARAMS,
    scratch_types=[
        pltpu.VMEM((6 * _CHUNKA,), jnp.float32),
        pltpu.VMEM((_CHUNKA,), jnp.int32),
        pltpu.VMEM((_CHUNKA,), jnp.float32),
        pltpu.VMEM((_TS * 7,), jnp.float32),
        pltpu.VMEM_SHARED((_SPAD * 7,), jnp.float32),
    ],
)


def _combine_body(parts_hbm, out_hbm, pa, pb, cbuf):
    cidx = lax.axis_index("c")
    sidx = lax.axis_index("s")
    w = sidx * _NC + cidx
    o0 = w * _TW
    for c in range(7):
        pltpu.sync_copy(parts_hbm.at[pl.ds(c * _SPAD + o0, _TW)],
                        pa.at[pl.ds(c * _TW, _TW)])
        pltpu.sync_copy(parts_hbm.at[pl.ds(_SPAD * 7 + c * _SPAD + o0, _TW)],
                        pb.at[pl.ds(c * _TW, _TW)])
    iota = lax.iota(jnp.int32, 16)

    def comb_body(j, carry):
        st = jnp.minimum(j * 16, _TW - 16)
        rows = (st + iota) * 8
        cnt = pa[pl.ds(6 * _TW + st, 16)] + pb[pl.ds(6 * _TW + st, 16)]
        inv = jnp.float32(1.0) / jnp.maximum(cnt, 1.0)
        for col in range(3):
            v = (pa[pl.ds(col * _TW + st, 16)]
                 + pb[pl.ds(col * _TW + st, 16)]) * inv
            plsc.store_scatter(cbuf, [rows + col], v)
        for col in range(3, 6):
            v = (pa[pl.ds(col * _TW + st, 16)]
                 + pb[pl.ds(col * _TW + st, 16)]) * (inv * _SIGMA)
            plsc.store_scatter(cbuf, [rows + col], v)
        plsc.store_scatter(cbuf, [rows + 6], inv)
        plsc.store_scatter(cbuf, [rows + 7], jnp.full((16,), 0.0, jnp.float32))
        return carry

    lax.fori_loop(0, -(-_TW // 16), comb_body, 0)
    pltpu.sync_copy(cbuf, out_hbm.at[pl.ds(w * _TW * 8, _TW * 8)])


_combine = pl.kernel(
    _combine_body,
    out_type=jax.ShapeDtypeStruct((_SPAD * 8,), jnp.float32),
    mesh=_MESH,
    compiler_params=_SC_PARAMS,
    scratch_types=[
        pltpu.VMEM((_TW * 7,), jnp.float32),
        pltpu.VMEM((_TW * 7,), jnp.float32),
        pltpu.VMEM((_TW * 8,), jnp.float32),
    ],
)


def _gather_body(tbl_hbm, idx_hbm, out_hbm, ibuf, gbuf, sbuf, table, sem):
    cidx = lax.axis_index("c")
    sidx = lax.axis_index("s")
    w = sidx * _NC + cidx
    t0 = sidx * _TS
    pltpu.sync_copy(tbl_hbm.at[pl.ds(t0, _TS)], sbuf)
    pltpu.sync_copy(sbuf, table.at[pl.ds(t0, _TS)])
    plsc.subcore_barrier()

    def chunk_body(k, carry):
        r0 = w * _RPW + k * _CHUNK
        pltpu.sync_copy(idx_hbm.at[pl.ds(r0, _CHUNK)], ibuf)
        pltpu.async_copy(table.at[ibuf.at[...]], gbuf, sem).wait()
        pltpu.sync_copy(gbuf, out_hbm.at[pl.ds(r0, _CHUNK)])
        return carry

    lax.fori_loop(0, _NCH, chunk_body, 0)


_gather = pl.kernel(
    _gather_body,
    out_type=jax.ShapeDtypeStruct((N, 8), jnp.float32),
    mesh=_MESH,
    compiler_params=_SC_PARAMS,
    scratch_types=[
        pltpu.VMEM((_CHUNK,), jnp.int32),
        pltpu.VMEM((_CHUNK, 8), jnp.float32),
        pltpu.VMEM((_TS, 8), jnp.float32),
        pltpu.VMEM_SHARED((_SPAD, 8), jnp.float32),
        pltpu.SemaphoreType.DMA,
    ],
)


def _dense_body(mol_ref, ex_ref, eh_ref, g_ref, w1_ref, w2_ref, r_ref, loss_ref):
    x3 = mol_ref[:, 0:3]
    xh = mol_ref[:, 3:19]
    com = g_ref[:, 0:3]
    m2 = g_ref[:, 3:6]
    invc = g_ref[:, 6:7]
    ex = ex_ref[...]
    tx = x3 - com
    zx = _ALPHA * tx + _SIGMA * ex - m2
    zh = (_ALPHA * np.float32(0.25)) * xh + _SIGMA * eh_ref[...]
    z = jnp.concatenate([zx, zh], axis=1)
    h = jnp.maximum(jnp.dot(z, w1_ref[...], preferred_element_type=jnp.float32), 0.0)
    e = jnp.dot(h, w2_ref[...], preferred_element_type=jnp.float32)
    e3 = e[:, 0:3]
    zhat = (np.float32(1.0) / _ALPHA) * zx - (_SIGMA / _ALPHA) * e3
    dr = tx - zhat
    r = jnp.sum(dr * dr, axis=1) * (invc[:, 0] * np.float32(1.0 / 3.0))
    dl = ex - e3
    l = jnp.sum(dl * dl, axis=1) * invc[:, 0]
    i = pl.program_id(0)
    valid = i * _BLK + lax.iota(jnp.int32, _BLK) < N
    r_ref[...] = jnp.where(valid, r, 0.0)
    l = jnp.where(valid, l, 0.0)

    @pl.when(i == 0)
    def _init():
        loss_ref[...] = jnp.zeros((1, 1), jnp.float32)

    loss_ref[...] += jnp.sum(l)[None, None]


def _dense_pass(mol_x, eps_x, eps_h, g, W1, W2):
    return pl.pallas_call(
        _dense_body,
        grid=(_NBLK,),
        in_specs=[
            pl.BlockSpec((_BLK, 19), lambda i: (i, 0)),
            pl.BlockSpec((_BLK, 3), lambda i: (i, 0)),
            pl.BlockSpec((_BLK, 16), lambda i: (i, 0)),
            pl.BlockSpec((_BLK, 8), lambda i: (i, 0)),
            pl.BlockSpec((19, 64), lambda i: (0, 0)),
            pl.BlockSpec((64, 19), lambda i: (0, 0)),
        ],
        out_specs=[
            pl.BlockSpec((_BLK,), lambda i: (i,)),
            pl.BlockSpec((1, 1), lambda i: (0, 0)),
        ],
        out_shape=[
            jax.ShapeDtypeStruct((_NPAD,), jnp.float32),
            jax.ShapeDtypeStruct((1, 1), jnp.float32),
        ],
    )(mol_x, eps_x, eps_h, g, W1, W2)


def _rscatter_body(r_hbm, idxp_hbm, out_hbm, rbuf, ibuf, zbuf, table):
    cidx = lax.axis_index("c")
    sidx = lax.axis_index("s")
    w = sidx * _NC + cidx
    t0 = sidx * _TS
    zer16 = jnp.full((16,), 0.0, jnp.float32)

    def fill_zeros(j, carry):
        zbuf[pl.ds(j * 16, 16)] = zer16
        return carry

    lax.fori_loop(0, _TS // 16, fill_zeros, 0)
    pltpu.sync_copy(zbuf, table.at[pl.ds(t0, _TS)])
    plsc.subcore_barrier()

    def chunk_body(k, carry):
        r0 = w * _RPWC + k * _CHUNKC
        pltpu.sync_copy(r_hbm.at[pl.ds(r0, _CHUNKC)], rbuf)
        pltpu.sync_copy(idxp_hbm.at[pl.ds(r0, _CHUNKC)], ibuf)
        pltpu.sync_copy(rbuf, table.at[ibuf.at[...]], add=True)
        return carry

    lax.fori_loop(0, _NCHC, chunk_body, 0)
    plsc.subcore_barrier()
    pltpu.sync_copy(table.at[pl.ds(t0, _TS)], zbuf)
    pltpu.sync_copy(zbuf, out_hbm.at[pl.ds(cidx * _SPAD + t0, _TS)])


_rscatter = pl.kernel(
    _rscatter_body,
    out_type=jax.ShapeDtypeStruct((2 * _SPAD,), jnp.float32),
    mesh=_MESH,
    compiler_params=_SC_PARAMS,
    scratch_types=[
        pltpu.VMEM((_CHUNKC,), jnp.float32),
        pltpu.VMEM((_CHUNKC,), jnp.int32),
        pltpu.VMEM((_TS,), jnp.float32),
        pltpu.VMEM_SHARED((_SPAD,), jnp.float32),
    ],
)


def _final_body(rp_ref, loss_ref, lo_ref, ro_ref):
    rsum = rp_ref[pl.ds(0, _SPAD)] + rp_ref[pl.ds(_SPAD, _SPAD)]
    ro_ref[...] = (jnp.sum(jnp.sqrt(rsum)) * np.float32(1.0 / S))[None, None]
    lo_ref[...] = loss_ref[...] * np.float32(1.0 / (6.0 * S))


def _final(rparts, loss_acc):
    return pl.pallas_call(
        _final_body,
        out_shape=[
            jax.ShapeDtypeStruct((1, 1), jnp.float32),
            jax.ShapeDtypeStruct((1, 1), jnp.float32),
        ],
    )(rparts, loss_acc)


def kernel(mol_x, mol_idx, pro_x, pro_idx, eps_x_mol, eps_h_mol, eps_h_pro, W1m, W2m, W1p, W2p):
    idx = mol_idx.astype(jnp.int32)
    idx_pad = jnp.concatenate([idx, jnp.zeros((_NPAD - N,), jnp.int32)])
    parts = _stats(mol_x[:, 0], mol_x[:, 1], mol_x[:, 2],
                   eps_x_mol[:, 0], eps_x_mol[:, 1], eps_x_mol[:, 2], idx)
    tbl = _combine(parts)
    g = _gather(tbl.reshape(_SPAD, 8), idx_pad)
    r, loss_acc = _dense_pass(mol_x, eps_x_mol, eps_h_mol, g, W1m, W2m)
    rparts = _rscatter(r, idx_pad)
    lo, ro = _final(rparts, loss_acc)
    return (lo[0, 0], ro[0, 0])


# speedup vs baseline: 3.0639x; 1.8540x over previous
import jax
import jax.numpy as jnp
import numpy as np
from jax import lax
from jax.experimental import pallas as pl
from jax.experimental.pallas import tpu as pltpu
from jax.experimental.pallas import tpu_sc as plsc

N = 800000
S = 50000

_T = np.float32(0.1)
_ALPHA2 = np.clip((np.float32(1.0) - _T * _T) ** np.float32(2.0), np.float32(1e-5), np.float32(1.0))
_ALPHA = np.sqrt(_ALPHA2).astype(np.float32)
_SIGMA = np.sqrt(np.float32(1.0) - _ALPHA2).astype(np.float32)

_NC, _NS = 2, 16
_NW = _NC * _NS
_RPW = N // _NW
_CHUNK = 1000
_NCH = _RPW // _CHUNK
_CHUNKA = 5000
_NCHA = _RPW // _CHUNKA

_SPAD = 50176
_TS = _SPAD // _NS
_TW = _SPAD // _NW

_BLK = 8192
_NBLK = -(-N // _BLK)
_NPAD = _NBLK * _BLK
_RPWC = _NPAD // _NW
_CHUNKC = 1568
_NCHC = _RPWC // _CHUNKC

_MESH = plsc.VectorSubcoreMesh(core_axis_name="c", subcore_axis_name="s")
_SC_PARAMS = pltpu.CompilerParams(needs_layout_passes=False,
                                  use_tc_tiling_on_sc=False)


def _stats_body(x0_hbm, x1_hbm, x2_hbm, e0_hbm, e1_hbm, e2_hbm, idx_hbm,
                out_hbm, cbufs, ibuf, onebuf, tbuf, table):
    cidx = lax.axis_index("c")
    sidx = lax.axis_index("s")
    w = sidx * _NC + cidx
    t0 = sidx * _TS * 7
    iota = lax.iota(jnp.int32, 16)
    ones16 = jnp.full((16,), 1.0, jnp.float32)
    zer16 = jnp.full((16,), 0.0, jnp.float32)

    def fill_ones(j, carry):
        onebuf[pl.ds(j * 16, 16)] = ones16
        return carry

    lax.fori_loop(0, _CHUNKA // 16 + 1, fill_ones, 0)

    def fill_zeros(j, carry):
        tbuf[pl.ds(j * 16, 16)] = zer16
        return carry

    lax.fori_loop(0, _TS * 7 // 16, fill_zeros, 0)
    pltpu.sync_copy(tbuf, table.at[pl.ds(t0, _TS * 7)])
    plsc.subcore_barrier()
    cols = (x0_hbm, x1_hbm, x2_hbm, e0_hbm, e1_hbm, e2_hbm)

    def chunk_body(k, carry):
        r0 = w * _RPW + k * _CHUNKA
        for c in range(6):
            pltpu.sync_copy(cols[c].at[pl.ds(r0, _CHUNKA)],
                            cbufs.at[pl.ds(c * _CHUNKA, _CHUNKA)])
        pltpu.sync_copy(idx_hbm.at[pl.ds(r0, _CHUNKA)], ibuf)
        for c in range(6):
            pltpu.sync_copy(cbufs.at[pl.ds(c * _CHUNKA, _CHUNKA)],
                            table.at[pl.ds(c * _SPAD, _SPAD)].at[ibuf.at[...]],
                            add=True)
        pltpu.sync_copy(onebuf,
                        table.at[pl.ds(6 * _SPAD, _SPAD)].at[ibuf.at[...]],
                        add=True)
        return carry

    lax.fori_loop(0, _NCHA, chunk_body, 0)
    plsc.subcore_barrier()
    pltpu.sync_copy(table.at[pl.ds(t0, _TS * 7)], tbuf)
    pltpu.sync_copy(tbuf, out_hbm.at[pl.ds(cidx * _SPAD * 7 + t0, _TS * 7)])


_stats = pl.kernel(
    _stats_body,
    out_type=jax.ShapeDtypeStruct((2 * _SPAD * 7,), jnp.float32),
    mesh=_MESH,
    compiler_params=_SC_PARAMS,
    scratch_types=[
        pltpu.VMEM((6 * _CHUNKA,), jnp.float32),
        pltpu.VMEM((_CHUNKA,), jnp.int32),
        pltpu.VMEM((_CHUNKA,), jnp.float32),
        pltpu.VMEM((_TS * 7,), jnp.float32),
        pltpu.VMEM_SHARED((_SPAD * 7,), jnp.float32),
    ],
)


def _combine_body(parts_hbm, out_hbm, pa, pb, cbuf):
    cidx = lax.axis_index("c")
    sidx = lax.axis_index("s")
    w = sidx * _NC + cidx
    o0 = w * _TW
    for c in range(7):
        pltpu.sync_copy(parts_hbm.at[pl.ds(c * _SPAD + o0, _TW)],
                        pa.at[pl.ds(c * _TW, _TW)])
        pltpu.sync_copy(parts_hbm.at[pl.ds(_SPAD * 7 + c * _SPAD + o0, _TW)],
                        pb.at[pl.ds(c * _TW, _TW)])

    def comb_body(j, carry):
        st = j * 16
        cnt = pa[pl.ds(6 * _TW + st, 16)] + pb[pl.ds(6 * _TW + st, 16)]
        inv = jnp.float32(1.0) / jnp.maximum(cnt, 1.0)
        for col in range(3):
            cbuf[pl.ds(col * _TW + st, 16)] = (
                pa[pl.ds(col * _TW + st, 16)]
                + pb[pl.ds(col * _TW + st, 16)]) * inv
        for col in range(3, 6):
            cbuf[pl.ds(col * _TW + st, 16)] = (
                pa[pl.ds(col * _TW + st, 16)]
                + pb[pl.ds(col * _TW + st, 16)]) * (inv * _SIGMA)
        cbuf[pl.ds(6 * _TW + st, 16)] = inv
        return carry

    lax.fori_loop(0, _TW // 16, comb_body, 0)
    for c in range(7):
        pltpu.sync_copy(cbuf.at[pl.ds(c * _TW, _TW)],
                        out_hbm.at[pl.ds(c * _SPAD + o0, _TW)])


_combine = pl.kernel(
    _combine_body,
    out_type=jax.ShapeDtypeStruct((_SPAD * 7,), jnp.float32),
    mesh=_MESH,
    compiler_params=_SC_PARAMS,
    scratch_types=[
        pltpu.VMEM((_TW * 7,), jnp.float32),
        pltpu.VMEM((_TW * 7,), jnp.float32),
        pltpu.VMEM((_TW * 7,), jnp.float32),
    ],
)


def _gather_body(tbl_hbm, idx_hbm, out_hbm, ibuf, gbuf, sbuf, table, sem):
    cidx = lax.axis_index("c")
    sidx = lax.axis_index("s")
    w = sidx * _NC + cidx
    t0 = sidx * _TS * 7
    pltpu.sync_copy(tbl_hbm.at[pl.ds(t0, _TS * 7)], sbuf)
    pltpu.sync_copy(sbuf, table.at[pl.ds(t0, _TS * 7)])
    plsc.subcore_barrier()

    def chunk_body(k, carry):
        r0 = w * _RPW + k * _CHUNKA
        pltpu.sync_copy(idx_hbm.at[pl.ds(r0, _CHUNKA)], ibuf)
        for c in range(7):
            pltpu.async_copy(
                table.at[pl.ds(c * _SPAD, _SPAD)].at[ibuf.at[...]],
                gbuf.at[pl.ds(c * _CHUNKA, _CHUNKA)], sem).wait()
        for c in range(7):
            pltpu.sync_copy(gbuf.at[pl.ds(c * _CHUNKA, _CHUNKA)],
                            out_hbm.at[pl.ds(c * N + r0, _CHUNKA)])
        return carry

    lax.fori_loop(0, _RPW // _CHUNKA, chunk_body, 0)


_gather = pl.kernel(
    _gather_body,
    out_type=jax.ShapeDtypeStruct((7 * N,), jnp.float32),
    mesh=_MESH,
    compiler_params=_SC_PARAMS,
    scratch_types=[
        pltpu.VMEM((_CHUNKA,), jnp.int32),
        pltpu.VMEM((7 * _CHUNKA,), jnp.float32),
        pltpu.VMEM((_TS * 7,), jnp.float32),
        pltpu.VMEM_SHARED((_SPAD * 7,), jnp.float32),
        pltpu.SemaphoreType.DMA,
    ],
)


def _dense_body(mol_ref, ex_ref, eh_ref, g_ref, w1t_ref, w2t_ref, r_ref, loss_ref):
    x3 = mol_ref[0:3, :]
    xh = mol_ref[3:19, :]
    com = g_ref[0:3, :]
    m2 = g_ref[3:6, :]
    invc = g_ref[6, :]
    ex = ex_ref[...]
    tx = x3 - com
    zx = _ALPHA * tx + _SIGMA * ex - m2
    zh = (_ALPHA * np.float32(0.25)) * xh + _SIGMA * eh_ref[...]
    z = jnp.concatenate([zx, zh], axis=0)
    h = jnp.maximum(jnp.dot(w1t_ref[...], z, preferred_element_type=jnp.float32), 0.0)
    e = jnp.dot(w2t_ref[...], h, preferred_element_type=jnp.float32)
    e3 = e[0:3, :]
    zhat = (np.float32(1.0) / _ALPHA) * zx - (_SIGMA / _ALPHA) * e3
    dr = tx - zhat
    r = jnp.sum(dr * dr, axis=0) * (invc * np.float32(1.0 / 3.0))
    dl = ex - e3
    l = jnp.sum(dl * dl, axis=0) * invc
    i = pl.program_id(0)
    valid = i * _BLK + lax.iota(jnp.int32, _BLK) < N
    r_ref[...] = jnp.where(valid, r, 0.0)
    l = jnp.where(valid, l, 0.0)

    @pl.when(i == 0)
    def _init():
        loss_ref[...] = jnp.zeros((1, 1), jnp.float32)

    loss_ref[...] += jnp.sum(l)[None, None]


def _dense_pass(mol_t, ex_t, eh_t, g, W1t, W2t):
    return pl.pallas_call(
        _dense_body,
        grid=(_NBLK,),
        in_specs=[
            pl.BlockSpec((19, _BLK), lambda i: (0, i)),
            pl.BlockSpec((3, _BLK), lambda i: (0, i)),
            pl.BlockSpec((16, _BLK), lambda i: (0, i)),
            pl.BlockSpec((7, _BLK), lambda i: (0, i)),
            pl.BlockSpec((64, 19), lambda i: (0, 0)),
            pl.BlockSpec((19, 64), lambda i: (0, 0)),
        ],
        out_specs=[
            pl.BlockSpec((_BLK,), lambda i: (i,)),
            pl.BlockSpec((1, 1), lambda i: (0, 0)),
        ],
        out_shape=[
            jax.ShapeDtypeStruct((_NPAD,), jnp.float32),
            jax.ShapeDtypeStruct((1, 1), jnp.float32),
        ],
    )(mol_t, ex_t, eh_t, g, W1t, W2t)


def _rscatter_body(r_hbm, idxp_hbm, out_hbm, rbuf, ibuf, zbuf, table):
    cidx = lax.axis_index("c")
    sidx = lax.axis_index("s")
    w = sidx * _NC + cidx
    t0 = sidx * _TS
    zer16 = jnp.full((16,), 0.0, jnp.float32)

    def fill_zeros(j, carry):
        zbuf[pl.ds(j * 16, 16)] = zer16
        return carry

    lax.fori_loop(0, _TS // 16, fill_zeros, 0)
    pltpu.sync_copy(zbuf, table.at[pl.ds(t0, _TS)])
    plsc.subcore_barrier()

    def chunk_body(k, carry):
        r0 = w * _RPWC + k * _CHUNKC
        pltpu.sync_copy(r_hbm.at[pl.ds(r0, _CHUNKC)], rbuf)
        pltpu.sync_copy(idxp_hbm.at[pl.ds(r0, _CHUNKC)], ibuf)
        pltpu.sync_copy(rbuf, table.at[ibuf.at[...]], add=True)
        return carry

    lax.fori_loop(0, _NCHC, chunk_body, 0)
    plsc.subcore_barrier()
    pltpu.sync_copy(table.at[pl.ds(t0, _TS)], zbuf)
    pltpu.sync_copy(zbuf, out_hbm.at[pl.ds(cidx * _SPAD + t0, _TS)])


_rscatter = pl.kernel(
    _rscatter_body,
    out_type=jax.ShapeDtypeStruct((2 * _SPAD,), jnp.float32),
    mesh=_MESH,
    compiler_params=_SC_PARAMS,
    scratch_types=[
        pltpu.VMEM((_CHUNKC,), jnp.float32),
        pltpu.VMEM((_CHUNKC,), jnp.int32),
        pltpu.VMEM((_TS,), jnp.float32),
        pltpu.VMEM_SHARED((_SPAD,), jnp.float32),
    ],
)


def _final_body(rp_ref, loss_ref, lo_ref, ro_ref):
    rsum = rp_ref[pl.ds(0, _SPAD)] + rp_ref[pl.ds(_SPAD, _SPAD)]
    ro_ref[...] = (jnp.sum(jnp.sqrt(rsum)) * np.float32(1.0 / S))[None, None]
    lo_ref[...] = loss_ref[...] * np.float32(1.0 / (6.0 * S))


def _final(rparts, loss_acc):
    return pl.pallas_call(
        _final_body,
        out_shape=[
            jax.ShapeDtypeStruct((1, 1), jnp.float32),
            jax.ShapeDtypeStruct((1, 1), jnp.float32),
        ],
    )(rparts, loss_acc)


def kernel(mol_x, mol_idx, pro_x, pro_idx, eps_x_mol, eps_h_mol, eps_h_pro, W1m, W2m, W1p, W2p):
    idx = mol_idx.astype(jnp.int32)
    idx_pad = jnp.concatenate([idx, jnp.zeros((_NPAD - N,), jnp.int32)])
    parts = _stats(mol_x[:, 0], mol_x[:, 1], mol_x[:, 2],
                   eps_x_mol[:, 0], eps_x_mol[:, 1], eps_x_mol[:, 2], idx)
    tbl = _combine(parts)
    g = _gather(tbl, idx)
    r, loss_acc = _dense_pass(mol_x.T, eps_x_mol.T, eps_h_mol.T,
                              g.reshape(7, N), W1m.T, W2m.T)
    rparts = _rscatter(r, idx_pad)
    lo, ro = _final(rparts, loss_acc)
    return (lo[0, 0], ro[0, 0])


# speedup vs baseline: 3.1687x; 1.0342x over previous
import jax
import jax.numpy as jnp
import numpy as np
from jax import lax
from jax.experimental import pallas as pl
from jax.experimental.pallas import tpu as pltpu
from jax.experimental.pallas import tpu_sc as plsc

N = 800000
S = 50000

_T = np.float32(0.1)
_ALPHA2 = np.clip((np.float32(1.0) - _T * _T) ** np.float32(2.0), np.float32(1e-5), np.float32(1.0))
_ALPHA = np.sqrt(_ALPHA2).astype(np.float32)
_SIGMA = np.sqrt(np.float32(1.0) - _ALPHA2).astype(np.float32)

_NC, _NS = 2, 16
_NW = _NC * _NS
_RPW = N // _NW
_CHUNK = 1000
_NCH = _RPW // _CHUNK
_CHUNKA = 5000
_NCHA = _RPW // _CHUNKA

_SPAD = 50176
_TS = _SPAD // _NS
_TW = _SPAD // _NW

_BLK = 16384
_NBLK = -(-N // _BLK)
_NPAD = _NBLK * _BLK
_RPWC = _NPAD // _NW
_CHUNKC = 1568
_NCHC = _RPWC // _CHUNKC

_MESH = plsc.VectorSubcoreMesh(core_axis_name="c", subcore_axis_name="s")
_SC_PARAMS = pltpu.CompilerParams(needs_layout_passes=False,
                                  use_tc_tiling_on_sc=False)


def _stats_body(x0_hbm, x1_hbm, x2_hbm, e0_hbm, e1_hbm, e2_hbm, idx_hbm,
                out_hbm, cbufs, ibuf, onebuf, tbuf, table):
    cidx = lax.axis_index("c")
    sidx = lax.axis_index("s")
    w = sidx * _NC + cidx
    t0 = sidx * _TS * 7
    iota = lax.iota(jnp.int32, 16)
    ones16 = jnp.full((16,), 1.0, jnp.float32)
    zer16 = jnp.full((16,), 0.0, jnp.float32)

    def fill_ones(j, carry):
        onebuf[pl.ds(j * 16, 16)] = ones16
        return carry

    lax.fori_loop(0, _CHUNKA // 16 + 1, fill_ones, 0)

    def fill_zeros(j, carry):
        tbuf[pl.ds(j * 16, 16)] = zer16
        return carry

    lax.fori_loop(0, _TS * 7 // 16, fill_zeros, 0)
    pltpu.sync_copy(tbuf, table.at[pl.ds(t0, _TS * 7)])
    plsc.subcore_barrier()
    cols = (x0_hbm, x1_hbm, x2_hbm, e0_hbm, e1_hbm, e2_hbm)

    def chunk_body(k, carry):
        r0 = w * _RPW + k * _CHUNKA
        for c in range(6):
            pltpu.sync_copy(cols[c].at[pl.ds(r0, _CHUNKA)],
                            cbufs.at[pl.ds(c * _CHUNKA, _CHUNKA)])
        pltpu.sync_copy(idx_hbm.at[pl.ds(r0, _CHUNKA)], ibuf)
        for c in range(6):
            pltpu.sync_copy(cbufs.at[pl.ds(c * _CHUNKA, _CHUNKA)],
                            table.at[pl.ds(c * _SPAD, _SPAD)].at[ibuf.at[...]],
                            add=True)
        pltpu.sync_copy(onebuf,
                        table.at[pl.ds(6 * _SPAD, _SPAD)].at[ibuf.at[...]],
                        add=True)
        return carry

    lax.fori_loop(0, _NCHA, chunk_body, 0)
    plsc.subcore_barrier()
    pltpu.sync_copy(table.at[pl.ds(t0, _TS * 7)], tbuf)
    pltpu.sync_copy(tbuf, out_hbm.at[pl.ds(cidx * _SPAD * 7 + t0, _TS * 7)])


_stats = pl.kernel(
    _stats_body,
    out_type=jax.ShapeDtypeStruct((2 * _SPAD * 7,), jnp.float32),
    mesh=_MESH,
    compiler_params=_SC_PARAMS,
    scratch_types=[
        pltpu.VMEM((6 * _CHUNKA,), jnp.float32),
        pltpu.VMEM((_CHUNKA,), jnp.int32),
        pltpu.VMEM((_CHUNKA,), jnp.float32),
        pltpu.VMEM((_TS * 7,), jnp.float32),
        pltpu.VMEM_SHARED((_SPAD * 7,), jnp.float32),
    ],
)


def _gather_body(parts_hbm, idx_hbm, out_hbm, pa, pb, cbuf, ibuf, gbuf, table, sem):
    cidx = lax.axis_index("c")
    sidx = lax.axis_index("s")
    w = sidx * _NC + cidx
    t0 = sidx * _TS
    for c in range(7):
        pltpu.sync_copy(parts_hbm.at[pl.ds(c * _SPAD + t0, _TS)],
                        pa.at[pl.ds(c * _TS, _TS)])
        pltpu.sync_copy(parts_hbm.at[pl.ds(_SPAD * 7 + c * _SPAD + t0, _TS)],
                        pb.at[pl.ds(c * _TS, _TS)])

    def comb_body(j, carry):
        st = j * 16
        cnt = pa[pl.ds(6 * _TS + st, 16)] + pb[pl.ds(6 * _TS + st, 16)]
        inv = jnp.float32(1.0) / jnp.maximum(cnt, 1.0)
        for col in range(3):
            cbuf[pl.ds(col * _TS + st, 16)] = (
                pa[pl.ds(col * _TS + st, 16)]
                + pb[pl.ds(col * _TS + st, 16)]) * inv
        for col in range(3, 6):
            cbuf[pl.ds(col * _TS + st, 16)] = (
                pa[pl.ds(col * _TS + st, 16)]
                + pb[pl.ds(col * _TS + st, 16)]) * (inv * _SIGMA)
        cbuf[pl.ds(6 * _TS + st, 16)] = inv
        return carry

    lax.fori_loop(0, _TS // 16, comb_body, 0)
    for c in range(7):
        pltpu.sync_copy(cbuf.at[pl.ds(c * _TS, _TS)],
                        table.at[pl.ds(c * _SPAD + t0, _TS)])
    plsc.subcore_barrier()

    def chunk_body(k, carry):
        r0 = w * _RPW + k * _CHUNKA
        pltpu.sync_copy(idx_hbm.at[pl.ds(r0, _CHUNKA)], ibuf)
        for c in range(7):
            pltpu.async_copy(
                table.at[pl.ds(c * _SPAD, _SPAD)].at[ibuf.at[...]],
                gbuf.at[pl.ds(c * _CHUNKA, _CHUNKA)], sem).wait()
        for c in range(7):
            pltpu.sync_copy(gbuf.at[pl.ds(c * _CHUNKA, _CHUNKA)],
                            out_hbm.at[pl.ds(c * N + r0, _CHUNKA)])
        return carry

    lax.fori_loop(0, _RPW // _CHUNKA, chunk_body, 0)


_gather = pl.kernel(
    _gather_body,
    out_type=jax.ShapeDtypeStruct((7 * N,), jnp.float32),
    mesh=_MESH,
    compiler_params=_SC_PARAMS,
    scratch_types=[
        pltpu.VMEM((7 * _TS,), jnp.float32),
        pltpu.VMEM((7 * _TS,), jnp.float32),
        pltpu.VMEM((7 * _TS,), jnp.float32),
        pltpu.VMEM((_CHUNKA,), jnp.int32),
        pltpu.VMEM((7 * _CHUNKA,), jnp.float32),
        pltpu.VMEM_SHARED((_SPAD * 7,), jnp.float32),
        pltpu.SemaphoreType.DMA,
    ],
)


def _dense_body(mol_ref, ex_ref, eh_ref, g_ref, w1t_ref, w2t_ref, r_ref, loss_ref):
    x3 = mol_ref[0:3, :]
    xh = mol_ref[3:19, :]
    com = g_ref[0:3, :]
    m2 = g_ref[3:6, :]
    invc = g_ref[6, :]
    ex = ex_ref[...]
    tx = x3 - com
    zx = _ALPHA * tx + _SIGMA * ex - m2
    zh = (_ALPHA * np.float32(0.25)) * xh + _SIGMA * eh_ref[...]
    z = jnp.concatenate([zx, zh], axis=0)
    h = jnp.maximum(jnp.dot(w1t_ref[...], z, preferred_element_type=jnp.float32), 0.0)
    e3 = jnp.dot(w2t_ref[...], h, preferred_element_type=jnp.float32)
    zhat = (np.float32(1.0) / _ALPHA) * zx - (_SIGMA / _ALPHA) * e3
    dr = tx - zhat
    r = jnp.sum(dr * dr, axis=0) * (invc * np.float32(1.0 / 3.0))
    dl = ex - e3
    l = jnp.sum(dl * dl, axis=0) * invc
    i = pl.program_id(0)
    valid = i * _BLK + lax.iota(jnp.int32, _BLK) < N
    r_ref[...] = jnp.where(valid, r, 0.0)
    l = jnp.where(valid, l, 0.0)

    @pl.when(i == 0)
    def _init():
        loss_ref[...] = jnp.zeros((1, 1), jnp.float32)

    loss_ref[...] += jnp.sum(l)[None, None]


def _dense_pass(mol_t, ex_t, eh_t, g, W1t, W2t):
    return pl.pallas_call(
        _dense_body,
        grid=(_NBLK,),
        in_specs=[
            pl.BlockSpec((19, _BLK), lambda i: (0, i)),
            pl.BlockSpec((3, _BLK), lambda i: (0, i)),
            pl.BlockSpec((16, _BLK), lambda i: (0, i)),
            pl.BlockSpec((7, _BLK), lambda i: (0, i)),
            pl.BlockSpec((64, 19), lambda i: (0, 0)),
            pl.BlockSpec((3, 64), lambda i: (0, 0)),
        ],
        out_specs=[
            pl.BlockSpec((_BLK,), lambda i: (i,)),
            pl.BlockSpec((1, 1), lambda i: (0, 0)),
        ],
        out_shape=[
            jax.ShapeDtypeStruct((_NPAD,), jnp.float32),
            jax.ShapeDtypeStruct((1, 1), jnp.float32),
        ],
    )(mol_t, ex_t, eh_t, g, W1t, W2t)


def _rscatter_body(r_hbm, idxp_hbm, out_hbm, rbuf, ibuf, zbuf, table):
    cidx = lax.axis_index("c")
    sidx = lax.axis_index("s")
    w = sidx * _NC + cidx
    t0 = sidx * _TS
    zer16 = jnp.full((16,), 0.0, jnp.float32)

    def fill_zeros(j, carry):
        zbuf[pl.ds(j * 16, 16)] = zer16
        return carry

    lax.fori_loop(0, _TS // 16, fill_zeros, 0)
    pltpu.sync_copy(zbuf, table.at[pl.ds(t0, _TS)])
    plsc.subcore_barrier()

    def chunk_body(k, carry):
        r0 = w * _RPWC + k * _CHUNKC
        pltpu.sync_copy(r_hbm.at[pl.ds(r0, _CHUNKC)], rbuf)
        pltpu.sync_copy(idxp_hbm.at[pl.ds(r0, _CHUNKC)], ibuf)
        pltpu.sync_copy(rbuf, table.at[ibuf.at[...]], add=True)
        return carry

    lax.fori_loop(0, _NCHC, chunk_body, 0)
    plsc.subcore_barrier()
    pltpu.sync_copy(table.at[pl.ds(t0, _TS)], zbuf)
    pltpu.sync_copy(zbuf, out_hbm.at[pl.ds(cidx * _SPAD + t0, _TS)])


_rscatter = pl.kernel(
    _rscatter_body,
    out_type=jax.ShapeDtypeStruct((2 * _SPAD,), jnp.float32),
    mesh=_MESH,
    compiler_params=_SC_PARAMS,
    scratch_types=[
        pltpu.VMEM((_CHUNKC,), jnp.float32),
        pltpu.VMEM((_CHUNKC,), jnp.int32),
        pltpu.VMEM((_TS,), jnp.float32),
        pltpu.VMEM_SHARED((_SPAD,), jnp.float32),
    ],
)


def _final_body(rp_ref, loss_ref, lo_ref, ro_ref):
    rsum = rp_ref[pl.ds(0, _SPAD)] + rp_ref[pl.ds(_SPAD, _SPAD)]
    ro_ref[...] = (jnp.sum(jnp.sqrt(rsum)) * np.float32(1.0 / S))[None, None]
    lo_ref[...] = loss_ref[...] * np.float32(1.0 / (6.0 * S))


def _final(rparts, loss_acc):
    return pl.pallas_call(
        _final_body,
        out_shape=[
            jax.ShapeDtypeStruct((1, 1), jnp.float32),
            jax.ShapeDtypeStruct((1, 1), jnp.float32),
        ],
    )(rparts, loss_acc)


def kernel(mol_x, mol_idx, pro_x, pro_idx, eps_x_mol, eps_h_mol, eps_h_pro, W1m, W2m, W1p, W2p):
    idx = mol_idx.astype(jnp.int32)
    idx_pad = jnp.concatenate([idx, jnp.zeros((_NPAD - N,), jnp.int32)])
    parts = _stats(mol_x[:, 0], mol_x[:, 1], mol_x[:, 2],
                   eps_x_mol[:, 0], eps_x_mol[:, 1], eps_x_mol[:, 2], idx)
    g = _gather(parts, idx)
    r, loss_acc = _dense_pass(mol_x.T, eps_x_mol.T, eps_h_mol.T,
                              g.reshape(7, N), W1m.T, W2m.T[:3])
    rparts = _rscatter(r, idx_pad)
    lo, ro = _final(rparts, loss_acc)
    return (lo[0, 0], ro[0, 0])


# speedup vs baseline: 7.3039x; 2.3050x over previous
import jax
import jax.numpy as jnp
import numpy as np
from jax import lax
from jax.experimental import pallas as pl
from jax.experimental.pallas import tpu as pltpu
from jax.experimental.pallas import tpu_sc as plsc

N = 800000
S = 50000

_T = np.float32(0.1)
_ALPHA2 = np.clip((np.float32(1.0) - _T * _T) ** np.float32(2.0), np.float32(1e-5), np.float32(1.0))
_ALPHA = np.sqrt(_ALPHA2).astype(np.float32)
_SIGMA = np.sqrt(np.float32(1.0) - _ALPHA2).astype(np.float32)

_NC, _NS = 2, 16
_NW = _NC * _NS
_RPW = N // _NW
_CHUNK = 1000
_NCH = _RPW // _CHUNK
_CHUNKA = 5000
_NCHA = _RPW // _CHUNKA

_SPAD = 50176
_TS = _SPAD // _NS
_TW = _SPAD // _NW

_BLK = 16384
_NBLK = -(-N // _BLK)
_NPAD = _NBLK * _BLK
_RPWC = _NPAD // _NW
_CHUNKC = 1568
_NCHC = _RPWC // _CHUNKC

_MESH = plsc.VectorSubcoreMesh(core_axis_name="c", subcore_axis_name="s")
_SC_PARAMS = pltpu.CompilerParams(needs_layout_passes=False,
                                  use_tc_tiling_on_sc=False)


def _stats_body(x0_hbm, x1_hbm, x2_hbm, e0_hbm, e1_hbm, e2_hbm, idx_hbm,
                out_hbm, cbufs, ibuf, onebuf, tbuf, table):
    cidx = lax.axis_index("c")
    sidx = lax.axis_index("s")
    w = sidx * _NC + cidx
    t0 = sidx * _TS * 7
    iota = lax.iota(jnp.int32, 16)
    ones16 = jnp.full((16,), 1.0, jnp.float32)
    zer16 = jnp.full((16,), 0.0, jnp.float32)

    def fill_ones(j, carry):
        onebuf[pl.ds(j * 16, 16)] = ones16
        return carry

    lax.fori_loop(0, _CHUNKA // 16 + 1, fill_ones, 0)

    def fill_zeros(j, carry):
        tbuf[pl.ds(j * 16, 16)] = zer16
        return carry

    lax.fori_loop(0, _TS * 7 // 16, fill_zeros, 0)
    pltpu.sync_copy(tbuf, table.at[pl.ds(t0, _TS * 7)])
    plsc.subcore_barrier()
    cols = (x0_hbm, x1_hbm, x2_hbm, e0_hbm, e1_hbm, e2_hbm)

    def chunk_body(k, carry):
        r0 = w * _RPW + k * _CHUNKA
        for c in range(6):
            pltpu.sync_copy(cols[c].at[pl.ds(r0, _CHUNKA)],
                            cbufs.at[pl.ds(c * _CHUNKA, _CHUNKA)])
        pltpu.sync_copy(idx_hbm.at[pl.ds(r0, _CHUNKA)], ibuf)
        for c in range(6):
            pltpu.sync_copy(cbufs.at[pl.ds(c * _CHUNKA, _CHUNKA)],
                            table.at[pl.ds(c * _SPAD, _SPAD)].at[ibuf.at[...]],
                            add=True)
        pltpu.sync_copy(onebuf,
                        table.at[pl.ds(6 * _SPAD, _SPAD)].at[ibuf.at[...]],
                        add=True)
        return carry

    lax.fori_loop(0, _NCHA, chunk_body, 0)
    plsc.subcore_barrier()
    pltpu.sync_copy(table.at[pl.ds(t0, _TS * 7)], tbuf)
    pltpu.sync_copy(tbuf, out_hbm.at[pl.ds(cidx * _SPAD * 7 + t0, _TS * 7)])


_stats = pl.kernel(
    _stats_body,
    out_type=jax.ShapeDtypeStruct((2 * _SPAD * 7,), jnp.float32),
    mesh=_MESH,
    compiler_params=_SC_PARAMS,
    scratch_types=[
        pltpu.VMEM((6 * _CHUNKA,), jnp.float32),
        pltpu.VMEM((_CHUNKA,), jnp.int32),
        pltpu.VMEM((_CHUNKA,), jnp.float32),
        pltpu.VMEM((_TS * 7,), jnp.float32),
        pltpu.VMEM_SHARED((_SPAD * 7,), jnp.float32),
    ],
)


def _gather_body(parts_hbm, idx_hbm, o0_hbm, o1_hbm, o2_hbm, o3_hbm, o4_hbm,
                 o5_hbm, o6_hbm, pa, pb, cbuf, ibuf, gbuf, table, sem):
    cidx = lax.axis_index("c")
    sidx = lax.axis_index("s")
    w = sidx * _NC + cidx
    t0 = sidx * _TS
    for c in range(7):
        pltpu.sync_copy(parts_hbm.at[pl.ds(c * _SPAD + t0, _TS)],
                        pa.at[pl.ds(c * _TS, _TS)])
        pltpu.sync_copy(parts_hbm.at[pl.ds(_SPAD * 7 + c * _SPAD + t0, _TS)],
                        pb.at[pl.ds(c * _TS, _TS)])

    def comb_body(j, carry):
        st = j * 16
        cnt = pa[pl.ds(6 * _TS + st, 16)] + pb[pl.ds(6 * _TS + st, 16)]
        inv = jnp.float32(1.0) / jnp.maximum(cnt, 1.0)
        for col in range(3):
            cbuf[pl.ds(col * _TS + st, 16)] = (
                pa[pl.ds(col * _TS + st, 16)]
                + pb[pl.ds(col * _TS + st, 16)]) * inv
        for col in range(3, 6):
            cbuf[pl.ds(col * _TS + st, 16)] = (
                pa[pl.ds(col * _TS + st, 16)]
                + pb[pl.ds(col * _TS + st, 16)]) * (inv * _SIGMA)
        cbuf[pl.ds(6 * _TS + st, 16)] = inv
        return carry

    lax.fori_loop(0, _TS // 16, comb_body, 0)
    for c in range(7):
        pltpu.sync_copy(cbuf.at[pl.ds(c * _TS, _TS)],
                        table.at[pl.ds(c * _SPAD + t0, _TS)])
    plsc.subcore_barrier()

    outs = (o0_hbm, o1_hbm, o2_hbm, o3_hbm, o4_hbm, o5_hbm, o6_hbm)

    def chunk_body(k, carry):
        r0 = w * _RPW + k * _CHUNKA
        pltpu.sync_copy(idx_hbm.at[pl.ds(r0, _CHUNKA)], ibuf)
        for c in range(7):
            pltpu.async_copy(
                table.at[pl.ds(c * _SPAD, _SPAD)].at[ibuf.at[...]],
                gbuf.at[pl.ds(c * _CHUNKA, _CHUNKA)], sem).wait()
        for c in range(7):
            pltpu.sync_copy(gbuf.at[pl.ds(c * _CHUNKA, _CHUNKA)],
                            outs[c].at[pl.ds(r0, _CHUNKA)])
        return carry

    lax.fori_loop(0, _RPW // _CHUNKA, chunk_body, 0)


_gather = pl.kernel(
    _gather_body,
    out_type=[jax.ShapeDtypeStruct((N,), jnp.float32)] * 7,
    mesh=_MESH,
    compiler_params=_SC_PARAMS,
    scratch_types=[
        pltpu.VMEM((7 * _TS,), jnp.float32),
        pltpu.VMEM((7 * _TS,), jnp.float32),
        pltpu.VMEM((7 * _TS,), jnp.float32),
        pltpu.VMEM((_CHUNKA,), jnp.int32),
        pltpu.VMEM((7 * _CHUNKA,), jnp.float32),
        pltpu.VMEM_SHARED((_SPAD * 7,), jnp.float32),
        pltpu.SemaphoreType.DMA,
    ],
)


def _dense_body(mol_ref, ex_ref, eh_ref, g0, g1, g2, g3, g4, g5, g6,
                w1t_ref, w2t_ref, r_ref, loss_ref):
    x3 = mol_ref[0:3, :]
    xh = mol_ref[3:19, :]
    com = jnp.stack([g0[...], g1[...], g2[...]], axis=0)
    m2 = jnp.stack([g3[...], g4[...], g5[...]], axis=0)
    invc = g6[...]
    ex = ex_ref[...]
    tx = x3 - com
    zx = _ALPHA * tx + _SIGMA * ex - m2
    zh = (_ALPHA * np.float32(0.25)) * xh + _SIGMA * eh_ref[...]
    z = jnp.concatenate([zx, zh], axis=0)
    h = jnp.maximum(jnp.dot(w1t_ref[...], z, preferred_element_type=jnp.float32), 0.0)
    e3 = jnp.dot(w2t_ref[...], h, preferred_element_type=jnp.float32)
    zhat = (np.float32(1.0) / _ALPHA) * zx - (_SIGMA / _ALPHA) * e3
    dr = tx - zhat
    r = jnp.sum(dr * dr, axis=0) * (invc * np.float32(1.0 / 3.0))
    dl = ex - e3
    l = jnp.sum(dl * dl, axis=0) * invc
    i = pl.program_id(0)
    valid = i * _BLK + lax.iota(jnp.int32, _BLK) < N
    r_ref[...] = jnp.where(valid, r, 0.0)
    l = jnp.where(valid, l, 0.0)

    @pl.when(i == 0)
    def _init():
        loss_ref[...] = jnp.zeros((1, 1), jnp.float32)

    loss_ref[...] += jnp.sum(l)[None, None]


def _dense_pass(mol_t, ex_t, eh_t, g, W1t, W2t):
    gspecs = [pl.BlockSpec((_BLK,), lambda i: (i,)) for _ in range(7)]
    return pl.pallas_call(
        _dense_body,
        grid=(_NBLK,),
        in_specs=[
            pl.BlockSpec((19, _BLK), lambda i: (0, i)),
            pl.BlockSpec((3, _BLK), lambda i: (0, i)),
            pl.BlockSpec((16, _BLK), lambda i: (0, i)),
            *gspecs,
            pl.BlockSpec((64, 19), lambda i: (0, 0)),
            pl.BlockSpec((3, 64), lambda i: (0, 0)),
        ],
        out_specs=[
            pl.BlockSpec((_BLK,), lambda i: (i,)),
            pl.BlockSpec((1, 1), lambda i: (0, 0)),
        ],
        out_shape=[
            jax.ShapeDtypeStruct((_NPAD,), jnp.float32),
            jax.ShapeDtypeStruct((1, 1), jnp.float32),
        ],
    )(mol_t, ex_t, eh_t, *g, W1t, W2t)


def _rscatter_body(r_hbm, idxp_hbm, out_hbm, rbuf, ibuf, zbuf, table):
    cidx = lax.axis_index("c")
    sidx = lax.axis_index("s")
    w = sidx * _NC + cidx
    t0 = sidx * _TS
    zer16 = jnp.full((16,), 0.0, jnp.float32)

    def fill_zeros(j, carry):
        zbuf[pl.ds(j * 16, 16)] = zer16
        return carry

    lax.fori_loop(0, _TS // 16, fill_zeros, 0)
    pltpu.sync_copy(zbuf, table.at[pl.ds(t0, _TS)])
    plsc.subcore_barrier()

    def chunk_body(k, carry):
        r0 = w * _RPWC + k * _CHUNKC
        pltpu.sync_copy(r_hbm.at[pl.ds(r0, _CHUNKC)], rbuf)
        pltpu.sync_copy(idxp_hbm.at[pl.ds(r0, _CHUNKC)], ibuf)
        pltpu.sync_copy(rbuf, table.at[ibuf.at[...]], add=True)
        return carry

    lax.fori_loop(0, _NCHC, chunk_body, 0)
    plsc.subcore_barrier()
    pltpu.sync_copy(table.at[pl.ds(t0, _TS)], zbuf)
    pltpu.sync_copy(zbuf, out_hbm.at[pl.ds(cidx * _SPAD + t0, _TS)])


_rscatter = pl.kernel(
    _rscatter_body,
    out_type=jax.ShapeDtypeStruct((2 * _SPAD,), jnp.float32),
    mesh=_MESH,
    compiler_params=_SC_PARAMS,
    scratch_types=[
        pltpu.VMEM((_CHUNKC,), jnp.float32),
        pltpu.VMEM((_CHUNKC,), jnp.int32),
        pltpu.VMEM((_TS,), jnp.float32),
        pltpu.VMEM_SHARED((_SPAD,), jnp.float32),
    ],
)


def _final_body(rp_ref, loss_ref, lo_ref, ro_ref):
    rsum = rp_ref[pl.ds(0, _SPAD)] + rp_ref[pl.ds(_SPAD, _SPAD)]
    ro_ref[...] = (jnp.sum(jnp.sqrt(rsum)) * np.float32(1.0 / S))[None, None]
    lo_ref[...] = loss_ref[...] * np.float32(1.0 / (6.0 * S))


def _final(rparts, loss_acc):
    return pl.pallas_call(
        _final_body,
        out_shape=[
            jax.ShapeDtypeStruct((1, 1), jnp.float32),
            jax.ShapeDtypeStruct((1, 1), jnp.float32),
        ],
    )(rparts, loss_acc)


def kernel(mol_x, mol_idx, pro_x, pro_idx, eps_x_mol, eps_h_mol, eps_h_pro, W1m, W2m, W1p, W2p):
    idx = mol_idx.astype(jnp.int32)
    idx_pad = jnp.concatenate([idx, jnp.zeros((_NPAD - N,), jnp.int32)])
    parts = _stats(mol_x[:, 0], mol_x[:, 1], mol_x[:, 2],
                   eps_x_mol[:, 0], eps_x_mol[:, 1], eps_x_mol[:, 2], idx)
    g = _gather(parts, idx)
    r, loss_acc = _dense_pass(mol_x.T, eps_x_mol.T, eps_h_mol.T,
                              g, W1m.T, W2m.T[:3])
    rparts = _rscatter(r, idx_pad)
    lo, ro = _final(rparts, loss_acc)
    return (lo[0, 0], ro[0, 0])


# speedup vs baseline: 7.6631x; 1.0492x over previous
import jax
import jax.numpy as jnp
import numpy as np
from jax import lax
from jax.experimental import pallas as pl
from jax.experimental.pallas import tpu as pltpu
from jax.experimental.pallas import tpu_sc as plsc

N = 800000
S = 50000

_T = np.float32(0.1)
_ALPHA2 = np.clip((np.float32(1.0) - _T * _T) ** np.float32(2.0), np.float32(1e-5), np.float32(1.0))
_ALPHA = np.sqrt(_ALPHA2).astype(np.float32)
_SIGMA = np.sqrt(np.float32(1.0) - _ALPHA2).astype(np.float32)

_NC, _NS = 2, 16
_NW = _NC * _NS
_RPW = N // _NW
_CHUNK = 1000
_NCH = _RPW // _CHUNK
_CHUNKA = 5000
_NCHA = _RPW // _CHUNKA

_SPAD = 50176
_TS = _SPAD // _NS
_TW = _SPAD // _NW

_BLK = 16384
_NBLK = -(-N // _BLK)
_NPAD = _NBLK * _BLK
_RPWC = _NPAD // _NW
_CHUNKC = 1568
_NCHC = _RPWC // _CHUNKC

_MESH = plsc.VectorSubcoreMesh(core_axis_name="c", subcore_axis_name="s")
_SC_PARAMS = pltpu.CompilerParams(needs_layout_passes=False,
                                  use_tc_tiling_on_sc=False)


def _stats_body(x0_hbm, x1_hbm, x2_hbm, e0_hbm, e1_hbm, e2_hbm, idx_hbm,
                out_hbm, cbufs, ibuf, onebuf, tbuf, table, sem):
    cidx = lax.axis_index("c")
    sidx = lax.axis_index("s")
    w = sidx * _NC + cidx
    t0 = sidx * _TS * 7
    iota = lax.iota(jnp.int32, 16)
    ones16 = jnp.full((16,), 1.0, jnp.float32)
    zer16 = jnp.full((16,), 0.0, jnp.float32)

    def fill_ones(j, carry):
        onebuf[pl.ds(j * 16, 16)] = ones16
        return carry

    lax.fori_loop(0, _CHUNKA // 16 + 1, fill_ones, 0)

    def fill_zeros(j, carry):
        tbuf[pl.ds(j * 16, 16)] = zer16
        return carry

    lax.fori_loop(0, _TS * 7 // 16, fill_zeros, 0)
    pltpu.sync_copy(tbuf, table.at[pl.ds(t0, _TS * 7)])
    plsc.subcore_barrier()
    cols = (x0_hbm, x1_hbm, x2_hbm, e0_hbm, e1_hbm, e2_hbm)

    def chunk_body(k, carry):
        r0 = w * _RPW + k * _CHUNKA
        descs = [pltpu.async_copy(cols[c].at[pl.ds(r0, _CHUNKA)],
                                  cbufs.at[pl.ds(c * _CHUNKA, _CHUNKA)], sem)
                 for c in range(6)]
        pltpu.sync_copy(idx_hbm.at[pl.ds(r0, _CHUNKA)], ibuf)
        for d in descs:
            d.wait()
        adds = [pltpu.async_copy(cbufs.at[pl.ds(c * _CHUNKA, _CHUNKA)],
                                 table.at[pl.ds(c * _SPAD, _SPAD)].at[ibuf.at[...]],
                                 sem, add=True)
                for c in range(6)]
        adds.append(pltpu.async_copy(
            onebuf, table.at[pl.ds(6 * _SPAD, _SPAD)].at[ibuf.at[...]],
            sem, add=True))
        for d in adds:
            d.wait()
        return carry

    lax.fori_loop(0, _NCHA, chunk_body, 0)
    plsc.subcore_barrier()
    pltpu.sync_copy(table.at[pl.ds(t0, _TS * 7)], tbuf)
    pltpu.sync_copy(tbuf, out_hbm.at[pl.ds(cidx * _SPAD * 7 + t0, _TS * 7)])


_stats = pl.kernel(
    _stats_body,
    out_type=jax.ShapeDtypeStruct((2 * _SPAD * 7,), jnp.float32),
    mesh=_MESH,
    compiler_params=_SC_PARAMS,
    scratch_types=[
        pltpu.VMEM((6 * _CHUNKA,), jnp.float32),
        pltpu.VMEM((_CHUNKA,), jnp.int32),
        pltpu.VMEM((_CHUNKA,), jnp.float32),
        pltpu.VMEM((_TS * 7,), jnp.float32),
        pltpu.VMEM_SHARED((_SPAD * 7,), jnp.float32),
        pltpu.SemaphoreType.DMA,
    ],
)


def _gather_body(parts_hbm, idx_hbm, o0_hbm, o1_hbm, o2_hbm, o3_hbm, o4_hbm,
                 o5_hbm, o6_hbm, pa, pb, cbuf, ibuf, gbuf, table, sem):
    cidx = lax.axis_index("c")
    sidx = lax.axis_index("s")
    w = sidx * _NC + cidx
    t0 = sidx * _TS
    for c in range(7):
        pltpu.sync_copy(parts_hbm.at[pl.ds(c * _SPAD + t0, _TS)],
                        pa.at[pl.ds(c * _TS, _TS)])
        pltpu.sync_copy(parts_hbm.at[pl.ds(_SPAD * 7 + c * _SPAD + t0, _TS)],
                        pb.at[pl.ds(c * _TS, _TS)])

    def comb_body(j, carry):
        st = j * 16
        cnt = pa[pl.ds(6 * _TS + st, 16)] + pb[pl.ds(6 * _TS + st, 16)]
        inv = jnp.float32(1.0) / jnp.maximum(cnt, 1.0)
        for col in range(3):
            cbuf[pl.ds(col * _TS + st, 16)] = (
                pa[pl.ds(col * _TS + st, 16)]
                + pb[pl.ds(col * _TS + st, 16)]) * inv
        for col in range(3, 6):
            cbuf[pl.ds(col * _TS + st, 16)] = (
                pa[pl.ds(col * _TS + st, 16)]
                + pb[pl.ds(col * _TS + st, 16)]) * (inv * _SIGMA)
        cbuf[pl.ds(6 * _TS + st, 16)] = inv
        return carry

    lax.fori_loop(0, _TS // 16, comb_body, 0)
    for c in range(7):
        pltpu.sync_copy(cbuf.at[pl.ds(c * _TS, _TS)],
                        table.at[pl.ds(c * _SPAD + t0, _TS)])
    plsc.subcore_barrier()

    outs = (o0_hbm, o1_hbm, o2_hbm, o3_hbm, o4_hbm, o5_hbm, o6_hbm)

    def chunk_body(k, carry):
        r0 = w * _RPW + k * _CHUNKA
        pltpu.sync_copy(idx_hbm.at[pl.ds(r0, _CHUNKA)], ibuf)
        gets = [pltpu.async_copy(
            table.at[pl.ds(c * _SPAD, _SPAD)].at[ibuf.at[...]],
            gbuf.at[pl.ds(c * _CHUNKA, _CHUNKA)], sem) for c in range(7)]
        for d in gets:
            d.wait()
        puts = [pltpu.async_copy(gbuf.at[pl.ds(c * _CHUNKA, _CHUNKA)],
                                 outs[c].at[pl.ds(r0, _CHUNKA)], sem)
                for c in range(7)]
        for d in puts:
            d.wait()
        return carry

    lax.fori_loop(0, _RPW // _CHUNKA, chunk_body, 0)


_gather = pl.kernel(
    _gather_body,
    out_type=[jax.ShapeDtypeStruct((N,), jnp.float32)] * 7,
    mesh=_MESH,
    compiler_params=_SC_PARAMS,
    scratch_types=[
        pltpu.VMEM((7 * _TS,), jnp.float32),
        pltpu.VMEM((7 * _TS,), jnp.float32),
        pltpu.VMEM((7 * _TS,), jnp.float32),
        pltpu.VMEM((_CHUNKA,), jnp.int32),
        pltpu.VMEM((7 * _CHUNKA,), jnp.float32),
        pltpu.VMEM_SHARED((_SPAD * 7,), jnp.float32),
        pltpu.SemaphoreType.DMA,
    ],
)


def _dense_body(mol_ref, ex_ref, eh_ref, g0, g1, g2, g3, g4, g5, g6,
                w1t_ref, w2t_ref, r_ref, loss_ref):
    x3 = mol_ref[0:3, :]
    xh = mol_ref[3:19, :]
    com = jnp.stack([g0[...], g1[...], g2[...]], axis=0)
    m2 = jnp.stack([g3[...], g4[...], g5[...]], axis=0)
    invc = g6[...]
    ex = ex_ref[...]
    tx = x3 - com
    zx = _ALPHA * tx + _SIGMA * ex - m2
    zh = (_ALPHA * np.float32(0.25)) * xh + _SIGMA * eh_ref[...]
    z = jnp.concatenate([zx, zh], axis=0)
    h = jnp.maximum(jnp.dot(w1t_ref[...], z, preferred_element_type=jnp.float32), 0.0)
    e3 = jnp.dot(w2t_ref[...], h, preferred_element_type=jnp.float32)
    zhat = (np.float32(1.0) / _ALPHA) * zx - (_SIGMA / _ALPHA) * e3
    dr = tx - zhat
    r = jnp.sum(dr * dr, axis=0) * (invc * np.float32(1.0 / 3.0))
    dl = ex - e3
    l = jnp.sum(dl * dl, axis=0) * invc
    i = pl.program_id(0)
    valid = i * _BLK + lax.iota(jnp.int32, _BLK) < N
    r_ref[...] = jnp.where(valid, r, 0.0)
    l = jnp.where(valid, l, 0.0)

    @pl.when(i == 0)
    def _init():
        loss_ref[...] = jnp.zeros((1, 1), jnp.float32)

    loss_ref[...] += jnp.sum(l)[None, None]


def _dense_pass(mol_t, ex_t, eh_t, g, W1t, W2t):
    gspecs = [pl.BlockSpec((_BLK,), lambda i: (i,)) for _ in range(7)]
    return pl.pallas_call(
        _dense_body,
        grid=(_NBLK,),
        in_specs=[
            pl.BlockSpec((19, _BLK), lambda i: (0, i)),
            pl.BlockSpec((3, _BLK), lambda i: (0, i)),
            pl.BlockSpec((16, _BLK), lambda i: (0, i)),
            *gspecs,
            pl.BlockSpec((64, 19), lambda i: (0, 0)),
            pl.BlockSpec((3, 64), lambda i: (0, 0)),
        ],
        out_specs=[
            pl.BlockSpec((_BLK,), lambda i: (i,)),
            pl.BlockSpec((1, 1), lambda i: (0, 0)),
        ],
        out_shape=[
            jax.ShapeDtypeStruct((_NPAD,), jnp.float32),
            jax.ShapeDtypeStruct((1, 1), jnp.float32),
        ],
    )(mol_t, ex_t, eh_t, *g, W1t, W2t)


def _rscatter_body(r_hbm, idxp_hbm, out_hbm, rbuf, ibuf, zbuf, table):
    cidx = lax.axis_index("c")
    sidx = lax.axis_index("s")
    w = sidx * _NC + cidx
    t0 = sidx * _TS
    zer16 = jnp.full((16,), 0.0, jnp.float32)

    def fill_zeros(j, carry):
        zbuf[pl.ds(j * 16, 16)] = zer16
        return carry

    lax.fori_loop(0, _TS // 16, fill_zeros, 0)
    pltpu.sync_copy(zbuf, table.at[pl.ds(t0, _TS)])
    plsc.subcore_barrier()

    def chunk_body(k, carry):
        r0 = w * _RPWC + k * _CHUNKC
        pltpu.sync_copy(r_hbm.at[pl.ds(r0, _CHUNKC)], rbuf)
        pltpu.sync_copy(idxp_hbm.at[pl.ds(r0, _CHUNKC)], ibuf)
        pltpu.sync_copy(rbuf, table.at[ibuf.at[...]], add=True)
        return carry

    lax.fori_loop(0, _NCHC, chunk_body, 0)
    plsc.subcore_barrier()
    pltpu.sync_copy(table.at[pl.ds(t0, _TS)], zbuf)
    pltpu.sync_copy(zbuf, out_hbm.at[pl.ds(cidx * _SPAD + t0, _TS)])


_rscatter = pl.kernel(
    _rscatter_body,
    out_type=jax.ShapeDtypeStruct((2 * _SPAD,), jnp.float32),
    mesh=_MESH,
    compiler_params=_SC_PARAMS,
    scratch_types=[
        pltpu.VMEM((_CHUNKC,), jnp.float32),
        pltpu.VMEM((_CHUNKC,), jnp.int32),
        pltpu.VMEM((_TS,), jnp.float32),
        pltpu.VMEM_SHARED((_SPAD,), jnp.float32),
    ],
)


def _final_body(rp_ref, loss_ref, lo_ref, ro_ref):
    rsum = rp_ref[pl.ds(0, _SPAD)] + rp_ref[pl.ds(_SPAD, _SPAD)]
    ro_ref[...] = (jnp.sum(jnp.sqrt(rsum)) * np.float32(1.0 / S))[None, None]
    lo_ref[...] = loss_ref[...] * np.float32(1.0 / (6.0 * S))


def _final(rparts, loss_acc):
    return pl.pallas_call(
        _final_body,
        out_shape=[
            jax.ShapeDtypeStruct((1, 1), jnp.float32),
            jax.ShapeDtypeStruct((1, 1), jnp.float32),
        ],
    )(rparts, loss_acc)


def kernel(mol_x, mol_idx, pro_x, pro_idx, eps_x_mol, eps_h_mol, eps_h_pro, W1m, W2m, W1p, W2p):
    idx = mol_idx.astype(jnp.int32)
    idx_pad = jnp.concatenate([idx, jnp.zeros((_NPAD - N,), jnp.int32)])
    parts = _stats(mol_x[:, 0], mol_x[:, 1], mol_x[:, 2],
                   eps_x_mol[:, 0], eps_x_mol[:, 1], eps_x_mol[:, 2], idx)
    g = _gather(parts, idx)
    r, loss_acc = _dense_pass(mol_x.T, eps_x_mol.T, eps_h_mol.T,
                              g, W1m.T, W2m.T[:3])
    rparts = _rscatter(r, idx_pad)
    lo, ro = _final(rparts, loss_acc)
    return (lo[0, 0], ro[0, 0])
